# baseline (speedup 1.0000x reference)
import functools

import jax
import jax.numpy as jnp
from jax import lax
from jax.experimental import pallas as pl
from jax.experimental.pallas import tpu as pltpu
from jax.experimental.pallas import tpu_sc as plsc

D = 128
H = 64
N = 10000
E = 320000
EP = E // 2
NL = 6
NSP = 3
CUTOFF = 6.0

NC = 2
NS = 16
CH = 80
CHP = CH // 2
EPT = E // NS
NCH = EPT // CH
NPH = 4
NH = 2504
NDR = NPH * NH
NDM = 8
ACCR = NH + NDM
RPT = 152
ZB = 152
ZT = ACCR - RPT * NS
DT = NH - RPT * NS

BN = 1000
BEP = 2000


def _init_body(x_ref, d_ref, emb_ref, h_ref, e_ref):
    i = pl.program_id(0)
    c = pl.program_id(1)

    @pl.when(c == 0)
    def _():
        xb = x_ref[...]
        acc = jnp.zeros((1, 1), jnp.float32)
        for sp in range(NSP):
            m = (xb == sp).astype(jnp.float32)
            acc = acc + m * emb_ref[sp:sp + 1, :]
        h_ref[...] = jnp.broadcast_to(acc, (BN, D))

    del i
    d = d_ref[...]
    n = (lax.broadcasted_iota(jnp.int32, (1, H), 1)
         + 1 + H * c).astype(jnp.float32)
    scale = jnp.float32((2.0 / CUTOFF) ** 0.5)
    w = jnp.float32(jnp.pi / CUTOFF)

    def bes(dcol):
        return scale * jnp.sin(n * (w * dcol)) / (dcol + 1e-8)

    e_ref[...] = jnp.concatenate([bes(d[:, 0:1]), bes(d[:, 1:2])],
                                 axis=1)[None]


def _tc_init(x, d2, emb):
    grid = (max(N // BN, EP // BEP), NC)
    return pl.pallas_call(
        _init_body,
        grid=grid,
        in_specs=[
            pl.BlockSpec((BN, 1), lambda i, c: (i % (N // BN), 0)),
            pl.BlockSpec((BEP, 2), lambda i, c: (i, 0)),
            pl.BlockSpec((NSP, D), lambda i, c: (0, 0)),
        ],
        out_specs=[
            pl.BlockSpec((BN, D), lambda i, c: (i % (N // BN), 0)),
            pl.BlockSpec((1, BEP, D), lambda i, c: (c, i, 0)),
        ],
        out_shape=[
            jax.ShapeDtypeStruct((N, D), jnp.float32),
            jax.ShapeDtypeStruct((NC, EP, D), jnp.float32),
        ],
    )(x, d2, emb)


def _tab_body(h_ref, wb_ref, wd_ref, we_ref, bb_ref, bd_ref, be_ref,
              wa_ref, ba_ref, ta_ref, te_ref, ha_ref):
    h = h_ref[...]
    hb = jnp.dot(h, wb_ref[0], preferred_element_type=jnp.float32) \
        + bb_ref[0]
    hd = jnp.dot(h, wd_ref[0], preferred_element_type=jnp.float32) \
        + bd_ref[0]
    he = jnp.dot(h, we_ref[...], preferred_element_type=jnp.float32) \
        + be_ref[...]
    ta_ref[...] = jnp.concatenate([hb, hd], axis=1)
    te_ref[...] = he
    ha_ref[...] = jnp.dot(h, wa_ref[...],
                          preferred_element_type=jnp.float32) + ba_ref[...]


def _tc_tab(h, WBi, WDi, WEi, bBi, bDi, bEi, WAi, bAi):
    grid = (N // BN, NC)
    wsplit = lambda w: jnp.stack([w[:, :H], w[:, H:]])
    bsplit = lambda b: b.reshape(NC, 1, H)
    return pl.pallas_call(
        _tab_body,
        grid=grid,
        in_specs=[
            pl.BlockSpec((BN, D), lambda i, c: (i, 0)),
            pl.BlockSpec((1, D, H), lambda i, c: (c, 0, 0)),
            pl.BlockSpec((1, D, H), lambda i, c: (c, 0, 0)),
            pl.BlockSpec((D, D), lambda i, c: (0, 0)),
            pl.BlockSpec((1, 1, H), lambda i, c: (c, 0, 0)),
            pl.BlockSpec((1, 1, H), lambda i, c: (c, 0, 0)),
            pl.BlockSpec((1, D), lambda i, c: (0, 0)),
            pl.BlockSpec((D, D), lambda i, c: (0, 0)),
            pl.BlockSpec((1, D), lambda i, c: (0, 0)),
        ],
        out_specs=[
            pl.BlockSpec((BN, D), lambda i, c: (c * (N // BN) + i, 0)),
            pl.BlockSpec((BN, D), lambda i, c: (i, 0)),
            pl.BlockSpec((BN, D), lambda i, c: (i, 0)),
        ],
        out_shape=[
            jax.ShapeDtypeStruct((NC * N, D), jnp.float32),
            jax.ShapeDtypeStruct((N, D), jnp.float32),
            jax.ShapeDtypeStruct((N, D), jnp.float32),
        ],
    )(h, wsplit(WBi), wsplit(WDi), WEi, bsplit(bBi), bsplit(bDi),
      bEi.reshape(1, D), WAi, bAi.reshape(1, D))


def _ec_body(e_ref, wa_ref, wb_ref, bc_ref, out_ref):
    p = e_ref[...]
    out = jnp.dot(p[0], wa_ref[0], preferred_element_type=jnp.float32) \
        + jnp.dot(p[1], wb_ref[0], preferred_element_type=jnp.float32) \
        + bc_ref[0]
    out_ref[...] = out[None]


def _tc_ec(e2p, WCAd, WCBd, bCp):
    grid = (EP // BEP, NC)
    return pl.pallas_call(
        _ec_body,
        grid=grid,
        in_specs=[
            pl.BlockSpec((NC, BEP, D), lambda i, c: (0, i, 0)),
            pl.BlockSpec((1, D, D), lambda i, c: (c, 0, 0)),
            pl.BlockSpec((1, D, D), lambda i, c: (c, 0, 0)),
            pl.BlockSpec((1, 1, D), lambda i, c: (c, 0, 0)),
        ],
        out_specs=pl.BlockSpec((1, BEP, D), lambda i, c: (c, i, 0)),
        out_shape=jax.ShapeDtypeStruct((NC, EP, D), jnp.float32),
    )(e2p, WCAd, WCBd, bCp)


def _upd_body(h_ref, ha_ref, nd_ref, out_ref):
    nd = nd_ref[...]
    num = jnp.concatenate([nd[0, :, 0:H], nd[1, :, 0:H]], axis=1)
    den = jnp.concatenate([nd[0, :, H:D], nd[1, :, H:D]], axis=1) + 1e-6
    v = ha_ref[...] + num / den
    sg = 1.0 / (1.0 + jnp.exp(-v))
    out_ref[...] = h_ref[...] + v * sg


def _tc_upd(h, hA, numden):
    grid = (N // BN,)
    return pl.pallas_call(
        _upd_body,
        grid=grid,
        in_specs=[
            pl.BlockSpec((BN, D), lambda i: (i, 0)),
            pl.BlockSpec((BN, D), lambda i: (i, 0)),
            pl.BlockSpec((NC, BN, D), lambda i: (0, i, 0)),
        ],
        out_specs=pl.BlockSpec((BN, D), lambda i: (i, 0)),
        out_shape=jax.ShapeDtypeStruct((N, D), jnp.float32),
    )(h, hA, numden)


def _final_body(h_ref, ha_ref, nd_ref, wh_ref, bh_ref, wo_ref, bo_ref,
                out_ref):
    nd = nd_ref[...]
    num = jnp.concatenate([nd[0, :, 0:H], nd[1, :, 0:H]], axis=1)
    den = jnp.concatenate([nd[0, :, H:D], nd[1, :, H:D]], axis=1) + 1e-6
    v = ha_ref[...] + num / den
    h = h_ref[...] + v * (1.0 / (1.0 + jnp.exp(-v)))
    t = jnp.dot(h, wh_ref[...], preferred_element_type=jnp.float32) \
        + bh_ref[...]
    t = t * (1.0 / (1.0 + jnp.exp(-t)))
    o = jnp.sum(t * wo_ref[...], axis=1, keepdims=True) + bo_ref[...]
    out_ref[...] = 1.0 / (1.0 + jnp.exp(-o))


def _tc_final(h, hA, numden, Wh, bh, Wo, bo):
    grid = (N // BN,)
    return pl.pallas_call(
        _final_body,
        grid=grid,
        in_specs=[
            pl.BlockSpec((BN, D), lambda i: (i, 0)),
            pl.BlockSpec((BN, D), lambda i: (i, 0)),
            pl.BlockSpec((NC, BN, D), lambda i: (0, i, 0)),
            pl.BlockSpec((D, D), lambda i: (0, 0)),
            pl.BlockSpec((1, D), lambda i: (0, 0)),
            pl.BlockSpec((1, D), lambda i: (0, 0)),
            pl.BlockSpec((1, 1), lambda i: (0, 0)),
        ],
        out_specs=pl.BlockSpec((BN, 1), lambda i: (i, 0)),
        out_shape=jax.ShapeDtypeStruct((N, 1), jnp.float32),
    )(h, hA, numden, Wh, bh.reshape(1, D), Wo.reshape(1, D),
      bo.reshape(1, 1))


def _sc_edge_body(e2p, ec2p, tabA, tabE, srcA2, dst2, s0v,
                  enew2p, numden,
                  sidx0, didx0, didxs0, ebuf0, ecbuf0, gabuf0, gebuf0,
                  msbuf0, sidx1, didx1, didxs1, ebuf1, ecbuf1, gabuf1,
                  gebuf1, msbuf1, zbuf, s0vm, acc,
                  semi0, semo0, semi1, semo1):
    c = lax.axis_index("c")
    s = lax.axis_index("s")

    pltpu.sync_copy(s0v, s0vm)
    lane = lax.broadcasted_iota(jnp.int32, (16,), 0)
    bnds = [jnp.int32(0),
            s0vm[pl.ds(0, 16)][0],
            s0vm[pl.ds(16, 16)][0],
            s0vm[pl.ds(32, 16)][0],
            jnp.int32(2 * E)]

    def tloc(v):
        return jnp.minimum(jnp.maximum(v - s * EPT, 0), EPT)

    def zero_acc():
        pltpu.sync_copy(zbuf, acc.at[pl.ds(pl.multiple_of(s * RPT, 8),
                                           RPT)])

        @pl.when(s == 0)
        def _():
            pltpu.sync_copy(zbuf.at[pl.ds(0, ZT)],
                            acc.at[pl.ds(pl.multiple_of(RPT * NS, 8), ZT)])

    def drain_acc(phase):
        soff = pl.multiple_of(s * RPT, 8)
        doff = pl.multiple_of(c * NDR + phase * NH + s * RPT, 8)
        pltpu.sync_copy(acc.at[pl.ds(soff, RPT)],
                        numden.at[pl.ds(doff, RPT)])

        @pl.when(s == 0)
        def _():
            toff = pl.multiple_of(RPT * NS, 8)
            pltpu.sync_copy(
                acc.at[pl.ds(toff, DT)],
                numden.at[pl.ds(pl.multiple_of(
                    c * NDR + phase * NH + RPT * NS, 8), DT)])

    def ioff(k):
        return pl.multiple_of(c * E + s * EPT + k * CH, 8)

    def poff(k):
        return pl.multiple_of(c * EP + s * (EPT // 2) + k * CHP, 8)

    sets = (
        (sidx0, didx0, didxs0, ebuf0, ecbuf0, gabuf0, gebuf0, msbuf0,
         semi0, semo0),
        (sidx1, didx1, didxs1, ebuf1, ecbuf1, gabuf1, gebuf1, msbuf1,
         semi1, semo1),
    )

    def load_idx(k, p):
        sx, dx = sets[p][0], sets[p][1]
        pltpu.sync_copy(srcA2.at[pl.ds(ioff(k), CH)], sx)
        pltpu.sync_copy(dst2.at[pl.ds(ioff(k), CH)], dx)

    def issue_data(k, p):
        sx, dx, _, eb, ecb, gab, geb, _, semi, _ = sets[p]
        pltpu.async_copy(tabA.at[sx], gab, semi)
        pltpu.async_copy(tabE.at[dx], geb, semi)
        pltpu.async_copy(e2p.at[pl.ds(poff(k), CHP)], eb, semi)
        pltpu.async_copy(ec2p.at[pl.ds(poff(k), CHP)], ecb, semi)

    def wait_data(p):
        sx, dx, _, eb, ecb, gab, geb, _, semi, _ = sets[p]
        pltpu.make_async_copy(tabA.at[sx], gab, semi).wait()
        pltpu.make_async_copy(tabE.at[dx], geb, semi).wait()
        pltpu.make_async_copy(e2p.at[pl.ds(0, CHP)], eb, semi).wait()
        pltpu.make_async_copy(ec2p.at[pl.ds(0, CHP)], ecb, semi).wait()

    def sync_stores(k, p):
        _, _, dxs, eb, _, _, _, msb, _, _ = sets[p]
        pltpu.sync_copy(msb, acc.at[dxs], add=True)
        pltpu.sync_copy(eb, enew2p.at[pl.ds(poff(k), CHP)])

    def compute(p, phase):
        _, dx, dxs, eb, ecb, gab, geb, msb, _, _ = sets[p]

        def edge(j2, carry2):
            for k8 in range(8):
                je = 2 * j2 + (k8 // 4)
                kf = k8 % 4
                sl_e = pl.ds(16 * k8, 16)
                sl_b = pl.ds(16 * kf, 16)
                sl_d = pl.ds(H + 16 * kf, 16)
                sl_g = pl.ds(H * c + 16 * kf, 16)
                ehat = ecb[j2, sl_e] + gab[je, sl_d] + geb[je, sl_g]
                sg = 1.0 / (1.0 + jnp.exp(-ehat))
                msb[je, sl_b] = sg * gab[je, sl_b]
                msb[je, sl_d] = sg
                eb[j2, sl_e] = eb[j2, sl_e] + ehat * sg
            return carry2
        lax.fori_loop(0, CHP, edge, 0)

        dummy = NH + (lane & (NDM - 1))
        qlo = phase * NH
        for t in range(CH // 16):
            v = dx[pl.ds(16 * t, 16)]
            r = v - qlo
            dxs[pl.ds(16 * t, 16)] = jnp.where((r >= 0) & (r < NH), r, dummy)

    def run_phase(lo, hi, phase):
        def par(k, fn):
            @pl.when((k & 1) == 0)
            def _():
                fn(0)

            @pl.when((k & 1) == 1)
            def _():
                fn(1)

        @pl.when(lo < hi)
        def _():
            def prolog(p):
                load_idx(lo, p)
                issue_data(lo, p)
            par(lo, prolog)

        def body(k, carry):
            def step(p):
                q = 1 - p

                @pl.when(k + 1 < hi)
                def _():
                    load_idx(k + 1, q)
                    issue_data(k + 1, q)
                wait_data(p)
                compute(p, phase)
                sync_stores(k, p)
            par(k, step)
            return carry
        lax.fori_loop(lo, hi, body, 0)

    def zrow(j, carry):
        for k in range(D // 16):
            zbuf[j, pl.ds(16 * k, 16)] = jnp.zeros((16,), jnp.float32)
        return carry
    lax.fori_loop(0, ZB, zrow, 0)

    for q in range(NPH):
        zero_acc()
        plsc.subcore_barrier()
        lo = tloc(bnds[q]) // CH
        hi = (tloc(bnds[q + 1]) + CH - 1) // CH
        run_phase(lo, hi, q)
        plsc.subcore_barrier()
        drain_acc(q)
        plsc.subcore_barrier()


@functools.lru_cache(maxsize=1)
def _sc_edge_kernel():
  return pl.kernel(
    _sc_edge_body,
    out_type=[
        jax.ShapeDtypeStruct((E, D), jnp.float32),
        jax.ShapeDtypeStruct((NC * NDR, D), jnp.float32),
    ],
    mesh=plsc.VectorSubcoreMesh(core_axis_name="c", subcore_axis_name="s",
                                num_cores=NC, num_subcores=NS),
    scratch_types=(
        [pltpu.VMEM((CH,), jnp.int32)] * 3
        + [pltpu.VMEM((CHP, D), jnp.float32)] * 2
        + [pltpu.VMEM((CH, D), jnp.float32)] * 3
        + [pltpu.VMEM((CH,), jnp.int32)] * 3
        + [pltpu.VMEM((CHP, D), jnp.float32)] * 2
        + [pltpu.VMEM((CH, D), jnp.float32)] * 3
        + [
            pltpu.VMEM((ZB, D), jnp.float32),
            pltpu.VMEM((48,), jnp.int32),
            pltpu.VMEM_SHARED((ACCR, D), jnp.float32),
            pltpu.SemaphoreType.DMA,
            pltpu.SemaphoreType.DMA,
            pltpu.SemaphoreType.DMA,
            pltpu.SemaphoreType.DMA,
        ]
    ),
  )


def _blkdiag(a):
    z = jnp.zeros((H, H), jnp.float32)
    return jnp.concatenate([jnp.concatenate([a, z], 1),
                            jnp.concatenate([z, a], 1)], 0)


def kernel(x, edge_index, edge_attr, emb, WA, bA, WB, bB, WC, bC, WD, bD,
           WE, bE, Wh, bh, Wo, bo):
    src = edge_index[0].astype(jnp.int32)
    dst = edge_index[1].astype(jnp.int32)

    grp = dst // NH
    pos = jnp.zeros((E,), jnp.int32)
    off = jnp.int32(0)
    bounds = []
    for q in range(NPH):
        m = (grp == q)
        cq = jnp.cumsum(m.astype(jnp.int32))
        pos = jnp.where(m, off + cq - 1, pos)
        off = off + cq[E - 1]
        bounds.append(off)
    perm = jnp.zeros((E,), jnp.int32).at[pos].set(
        jnp.arange(E, dtype=jnp.int32))
    src = jnp.take(src, perm)
    dst = jnp.take(dst, perm)
    attr = jnp.take(edge_attr, perm)

    srcA2 = jnp.concatenate([src, src + N])
    dst2 = jnp.concatenate([dst, dst])
    s0v = jnp.concatenate([jnp.full((16,), b, jnp.int32)
                           for b in bounds[:3]])

    h, e2p = _tc_init(x.reshape(N, 1).astype(jnp.int32),
                      attr.reshape(EP, 2), emb)

    for i in range(NL):
        tabA, tabE, hA = _tc_tab(h, WB[i], WD[i], WE[i], bB[i], bD[i],
                                 bE[i], WA[i], bA[i])
        WCAd = jnp.stack([_blkdiag(WC[i][0:H, 0:H]),
                          _blkdiag(WC[i][0:H, H:D])])
        WCBd = jnp.stack([_blkdiag(WC[i][H:D, 0:H]),
                          _blkdiag(WC[i][H:D, H:D])])
        bCp = jnp.concatenate([bC[i].reshape(NC, H)] * 2,
                              axis=1).reshape(NC, 1, D)
        ec2p = _tc_ec(e2p, WCAd, WCBd, bCp)

        enew, numden = _sc_edge_kernel()(e2p.reshape(E, D),
                                         ec2p.reshape(E, D),
                                         tabA, tabE, srcA2, dst2, s0v)
        e2p = enew.reshape(NC, EP, D)
        nd = numden.reshape(NC, NDR, D)
        if i < NL - 1:
            h = _tc_upd(h, hA, nd)
        else:
            out = _tc_final(h, hA, nd, Wh, bh, Wo, bo)
    return out


# speedup vs baseline: 1.0009x; 1.0009x over previous
import functools

import jax
import jax.numpy as jnp
from jax import lax
from jax.experimental import pallas as pl
from jax.experimental.pallas import tpu as pltpu
from jax.experimental.pallas import tpu_sc as plsc

D = 128
H = 64
N = 10000
E = 320000
EP = E // 2
NL = 6
NSP = 3
CUTOFF = 6.0

NC = 2
NS = 16
CH = 80
CHP = CH // 2
EPT = E // NS
NCH = EPT // CH
NPH = 4
NH = 2504
NDR = NPH * NH
NDM = 8
ACCR = NH + NDM
RPT = 152
ZB = 152
ZT = ACCR - RPT * NS
DT = NH - RPT * NS

BN = 1000
BEP = 2000


def _init_body(x_ref, d_ref, emb_ref, h_ref, e_ref):
    i = pl.program_id(0)
    c = pl.program_id(1)

    @pl.when(c == 0)
    def _():
        xb = x_ref[...]
        acc = jnp.zeros((1, 1), jnp.float32)
        for sp in range(NSP):
            m = (xb == sp).astype(jnp.float32)
            acc = acc + m * emb_ref[sp:sp + 1, :]
        h_ref[...] = jnp.broadcast_to(acc, (BN, D))

    del i
    d = d_ref[...]
    n = (lax.broadcasted_iota(jnp.int32, (1, H), 1)
         + 1 + H * c).astype(jnp.float32)
    scale = jnp.float32((2.0 / CUTOFF) ** 0.5)
    w = jnp.float32(jnp.pi / CUTOFF)

    def bes(dcol):
        return scale * jnp.sin(n * (w * dcol)) / (dcol + 1e-8)

    e_ref[...] = jnp.concatenate([bes(d[:, 0:1]), bes(d[:, 1:2])],
                                 axis=1)[None]


def _tc_init(x, d2, emb):
    grid = (max(N // BN, EP // BEP), NC)
    return pl.pallas_call(
        _init_body,
        grid=grid,
        in_specs=[
            pl.BlockSpec((BN, 1), lambda i, c: (i % (N // BN), 0)),
            pl.BlockSpec((BEP, 2), lambda i, c: (i, 0)),
            pl.BlockSpec((NSP, D), lambda i, c: (0, 0)),
        ],
        out_specs=[
            pl.BlockSpec((BN, D), lambda i, c: (i % (N // BN), 0)),
            pl.BlockSpec((1, BEP, D), lambda i, c: (c, i, 0)),
        ],
        out_shape=[
            jax.ShapeDtypeStruct((N, D), jnp.float32),
            jax.ShapeDtypeStruct((NC, EP, D), jnp.float32),
        ],
    )(x, d2, emb)


def _tab_body(h_ref, wb_ref, wd_ref, we_ref, bb_ref, bd_ref, be_ref,
              wa_ref, ba_ref, ta_ref, te_ref, ha_ref):
    h = h_ref[...]
    hb = jnp.dot(h, wb_ref[0], preferred_element_type=jnp.float32) \
        + bb_ref[0]
    hd = jnp.dot(h, wd_ref[0], preferred_element_type=jnp.float32) \
        + bd_ref[0]
    he = jnp.dot(h, we_ref[...], preferred_element_type=jnp.float32) \
        + be_ref[...]
    ta_ref[...] = jnp.concatenate([hb, hd], axis=1)
    te_ref[...] = he
    ha_ref[...] = jnp.dot(h, wa_ref[...],
                          preferred_element_type=jnp.float32) + ba_ref[...]


def _tc_tab(h, WBi, WDi, WEi, bBi, bDi, bEi, WAi, bAi):
    grid = (N // BN, NC)
    wsplit = lambda w: jnp.stack([w[:, :H], w[:, H:]])
    bsplit = lambda b: b.reshape(NC, 1, H)
    return pl.pallas_call(
        _tab_body,
        grid=grid,
        in_specs=[
            pl.BlockSpec((BN, D), lambda i, c: (i, 0)),
            pl.BlockSpec((1, D, H), lambda i, c: (c, 0, 0)),
            pl.BlockSpec((1, D, H), lambda i, c: (c, 0, 0)),
            pl.BlockSpec((D, D), lambda i, c: (0, 0)),
            pl.BlockSpec((1, 1, H), lambda i, c: (c, 0, 0)),
            pl.BlockSpec((1, 1, H), lambda i, c: (c, 0, 0)),
            pl.BlockSpec((1, D), lambda i, c: (0, 0)),
            pl.BlockSpec((D, D), lambda i, c: (0, 0)),
            pl.BlockSpec((1, D), lambda i, c: (0, 0)),
        ],
        out_specs=[
            pl.BlockSpec((BN, D), lambda i, c: (c * (N // BN) + i, 0)),
            pl.BlockSpec((BN, D), lambda i, c: (i, 0)),
            pl.BlockSpec((BN, D), lambda i, c: (i, 0)),
        ],
        out_shape=[
            jax.ShapeDtypeStruct((NC * N, D), jnp.float32),
            jax.ShapeDtypeStruct((N, D), jnp.float32),
            jax.ShapeDtypeStruct((N, D), jnp.float32),
        ],
    )(h, wsplit(WBi), wsplit(WDi), WEi, bsplit(bBi), bsplit(bDi),
      bEi.reshape(1, D), WAi, bAi.reshape(1, D))


def _ec_body(e_ref, wa_ref, wb_ref, bc_ref, out_ref):
    p = e_ref[...]
    out = jnp.dot(p[0], wa_ref[0], preferred_element_type=jnp.float32) \
        + jnp.dot(p[1], wb_ref[0], preferred_element_type=jnp.float32) \
        + bc_ref[0]
    out_ref[...] = out[None]


def _tc_ec(e2p, WCAd, WCBd, bCp):
    grid = (EP // BEP, NC)
    return pl.pallas_call(
        _ec_body,
        grid=grid,
        in_specs=[
            pl.BlockSpec((NC, BEP, D), lambda i, c: (0, i, 0)),
            pl.BlockSpec((1, D, D), lambda i, c: (c, 0, 0)),
            pl.BlockSpec((1, D, D), lambda i, c: (c, 0, 0)),
            pl.BlockSpec((1, 1, D), lambda i, c: (c, 0, 0)),
        ],
        out_specs=pl.BlockSpec((1, BEP, D), lambda i, c: (c, i, 0)),
        out_shape=jax.ShapeDtypeStruct((NC, EP, D), jnp.float32),
    )(e2p, WCAd, WCBd, bCp)


def _upd_body(h_ref, ha_ref, nd_ref, out_ref):
    nd = nd_ref[...]
    num = jnp.concatenate([nd[0, :, 0:H], nd[1, :, 0:H]], axis=1)
    den = jnp.concatenate([nd[0, :, H:D], nd[1, :, H:D]], axis=1) + 1e-6
    v = ha_ref[...] + num / den
    sg = 1.0 / (1.0 + jnp.exp(-v))
    out_ref[...] = h_ref[...] + v * sg


def _tc_upd(h, hA, numden):
    grid = (N // BN,)
    return pl.pallas_call(
        _upd_body,
        grid=grid,
        in_specs=[
            pl.BlockSpec((BN, D), lambda i: (i, 0)),
            pl.BlockSpec((BN, D), lambda i: (i, 0)),
            pl.BlockSpec((NC, BN, D), lambda i: (0, i, 0)),
        ],
        out_specs=pl.BlockSpec((BN, D), lambda i: (i, 0)),
        out_shape=jax.ShapeDtypeStruct((N, D), jnp.float32),
    )(h, hA, numden)


def _final_body(h_ref, ha_ref, nd_ref, wh_ref, bh_ref, wo_ref, bo_ref,
                out_ref):
    nd = nd_ref[...]
    num = jnp.concatenate([nd[0, :, 0:H], nd[1, :, 0:H]], axis=1)
    den = jnp.concatenate([nd[0, :, H:D], nd[1, :, H:D]], axis=1) + 1e-6
    v = ha_ref[...] + num / den
    h = h_ref[...] + v * (1.0 / (1.0 + jnp.exp(-v)))
    t = jnp.dot(h, wh_ref[...], preferred_element_type=jnp.float32) \
        + bh_ref[...]
    t = t * (1.0 / (1.0 + jnp.exp(-t)))
    o = jnp.sum(t * wo_ref[...], axis=1, keepdims=True) + bo_ref[...]
    out_ref[...] = 1.0 / (1.0 + jnp.exp(-o))


def _tc_final(h, hA, numden, Wh, bh, Wo, bo):
    grid = (N // BN,)
    return pl.pallas_call(
        _final_body,
        grid=grid,
        in_specs=[
            pl.BlockSpec((BN, D), lambda i: (i, 0)),
            pl.BlockSpec((BN, D), lambda i: (i, 0)),
            pl.BlockSpec((NC, BN, D), lambda i: (0, i, 0)),
            pl.BlockSpec((D, D), lambda i: (0, 0)),
            pl.BlockSpec((1, D), lambda i: (0, 0)),
            pl.BlockSpec((1, D), lambda i: (0, 0)),
            pl.BlockSpec((1, 1), lambda i: (0, 0)),
        ],
        out_specs=pl.BlockSpec((BN, 1), lambda i: (i, 0)),
        out_shape=jax.ShapeDtypeStruct((N, 1), jnp.float32),
    )(h, hA, numden, Wh, bh.reshape(1, D), Wo.reshape(1, D),
      bo.reshape(1, 1))


def _sc_edge_body(e2p, ec2p, tabA, tabE, srcA2, dst2, s0v,
                  enew2p, numden,
                  sidx, didx, didxs, ebuf, ecbuf, gabuf, gebuf,
                  msbuf, zbuf, s0vm, acc, semi):
    c = lax.axis_index("c")
    s = lax.axis_index("s")

    pltpu.sync_copy(s0v, s0vm)
    lane = lax.broadcasted_iota(jnp.int32, (16,), 0)
    bnds = [jnp.int32(0),
            s0vm[pl.ds(0, 16)][0],
            s0vm[pl.ds(16, 16)][0],
            s0vm[pl.ds(32, 16)][0],
            jnp.int32(2 * E)]

    def tloc(v):
        return jnp.minimum(jnp.maximum(v - s * EPT, 0), EPT)

    def zero_acc():
        pltpu.sync_copy(zbuf, acc.at[pl.ds(pl.multiple_of(s * RPT, 8),
                                           RPT)])

        @pl.when(s == 0)
        def _():
            pltpu.sync_copy(zbuf.at[pl.ds(0, ZT)],
                            acc.at[pl.ds(pl.multiple_of(RPT * NS, 8), ZT)])

    def drain_acc(phase):
        soff = pl.multiple_of(s * RPT, 8)
        doff = pl.multiple_of(c * NDR + phase * NH + s * RPT, 8)
        pltpu.sync_copy(acc.at[pl.ds(soff, RPT)],
                        numden.at[pl.ds(doff, RPT)])

        @pl.when(s == 0)
        def _():
            toff = pl.multiple_of(RPT * NS, 8)
            pltpu.sync_copy(
                acc.at[pl.ds(toff, DT)],
                numden.at[pl.ds(pl.multiple_of(
                    c * NDR + phase * NH + RPT * NS, 8), DT)])

    def ioff(k):
        return pl.multiple_of(c * E + s * EPT + k * CH, 8)

    def poff(k):
        return pl.multiple_of(c * EP + s * (EPT // 2) + k * CHP, 8)

    def load_idx(k, p):
        pltpu.sync_copy(srcA2.at[pl.ds(ioff(k), CH)], sidx.at[p])
        pltpu.sync_copy(dst2.at[pl.ds(ioff(k), CH)], didx.at[p])

    def issue_data(k, p):
        pltpu.async_copy(tabA.at[sidx.at[p]], gabuf.at[pl.ds(p * CH, CH)],
                         semi)
        pltpu.async_copy(tabE.at[didx.at[p]], gebuf.at[pl.ds(p * CH, CH)],
                         semi)
        pltpu.async_copy(e2p.at[pl.ds(poff(k), CHP)],
                         ebuf.at[pl.ds(p * CHP, CHP)], semi)
        pltpu.async_copy(ec2p.at[pl.ds(poff(k), CHP)],
                         ecbuf.at[pl.ds(p * CHP, CHP)], semi)

    def wait_data(p):
        pltpu.make_async_copy(tabA.at[sidx.at[p]],
                              gabuf.at[pl.ds(p * CH, CH)], semi).wait()
        pltpu.make_async_copy(tabE.at[didx.at[p]],
                              gebuf.at[pl.ds(p * CH, CH)], semi).wait()
        pltpu.make_async_copy(e2p.at[pl.ds(0, CHP)],
                              ebuf.at[pl.ds(p * CHP, CHP)], semi).wait()
        pltpu.make_async_copy(ec2p.at[pl.ds(0, CHP)],
                              ecbuf.at[pl.ds(p * CHP, CHP)], semi).wait()

    def sync_stores(k, p):
        pltpu.sync_copy(msbuf.at[pl.ds(p * CH, CH)], acc.at[didxs.at[p]],
                        add=True)
        pltpu.sync_copy(ebuf.at[pl.ds(p * CHP, CHP)],
                        enew2p.at[pl.ds(poff(k), CHP)])

    def compute(p, phase):
        eo = p * CHP
        go = p * CH

        def edge(j2, carry2):
            for k8 in range(8):
                je = go + 2 * j2 + (k8 // 4)
                kf = k8 % 4
                sl_e = pl.ds(16 * k8, 16)
                sl_b = pl.ds(16 * kf, 16)
                sl_d = pl.ds(H + 16 * kf, 16)
                sl_g = pl.ds(H * c + 16 * kf, 16)
                ehat = ecbuf[eo + j2, sl_e] + gabuf[je, sl_d] \
                    + gebuf[je, sl_g]
                sg = 1.0 / (1.0 + jnp.exp(-ehat))
                msbuf[je, sl_b] = sg * gabuf[je, sl_b]
                msbuf[je, sl_d] = sg
                ebuf[eo + j2, sl_e] = ebuf[eo + j2, sl_e] + ehat * sg
            return carry2
        lax.fori_loop(0, CHP, edge, 0)

        dummy = NH + (lane & (NDM - 1))
        qlo = phase * NH
        for t in range(CH // 16):
            v = didx[p, pl.ds(16 * t, 16)]
            r = v - qlo
            didxs[p, pl.ds(16 * t, 16)] = jnp.where((r >= 0) & (r < NH), r,
                                                    dummy)

    def run_phase(lo, hi, phase):
        @pl.when(lo < hi)
        def _():
            load_idx(lo, lo & 1)
            issue_data(lo, lo & 1)

        def body(k, carry):
            p = k & 1
            wait_data(p)

            @pl.when(k + 1 < hi)
            def _():
                load_idx(k + 1, 1 - p)
                issue_data(k + 1, 1 - p)
            compute(p, phase)
            sync_stores(k, p)
            return carry
        lax.fori_loop(lo, hi, body, 0)

    def zrow(j, carry):
        for k in range(D // 16):
            zbuf[j, pl.ds(16 * k, 16)] = jnp.zeros((16,), jnp.float32)
        return carry
    lax.fori_loop(0, ZB, zrow, 0)

    for q in range(NPH):
        zero_acc()
        plsc.subcore_barrier()
        lo = tloc(bnds[q]) // CH
        hi = (tloc(bnds[q + 1]) + CH - 1) // CH
        run_phase(lo, hi, q)
        plsc.subcore_barrier()
        drain_acc(q)
        plsc.subcore_barrier()


@functools.lru_cache(maxsize=1)
def _sc_edge_kernel():
  return pl.kernel(
    _sc_edge_body,
    out_type=[
        jax.ShapeDtypeStruct((E, D), jnp.float32),
        jax.ShapeDtypeStruct((NC * NDR, D), jnp.float32),
    ],
    mesh=plsc.VectorSubcoreMesh(core_axis_name="c", subcore_axis_name="s",
                                num_cores=NC, num_subcores=NS),
    scratch_types=[
        pltpu.VMEM((2, CH), jnp.int32),
        pltpu.VMEM((2, CH), jnp.int32),
        pltpu.VMEM((2, CH), jnp.int32),
        pltpu.VMEM((2 * CHP, D), jnp.float32),
        pltpu.VMEM((2 * CHP, D), jnp.float32),
        pltpu.VMEM((2 * CH, D), jnp.float32),
        pltpu.VMEM((2 * CH, D), jnp.float32),
        pltpu.VMEM((2 * CH, D), jnp.float32),
        pltpu.VMEM((ZB, D), jnp.float32),
        pltpu.VMEM((48,), jnp.int32),
        pltpu.VMEM_SHARED((ACCR, D), jnp.float32),
        pltpu.SemaphoreType.DMA,
    ],
  )


def _blkdiag(a):
    z = jnp.zeros((H, H), jnp.float32)
    return jnp.concatenate([jnp.concatenate([a, z], 1),
                            jnp.concatenate([z, a], 1)], 0)


def kernel(x, edge_index, edge_attr, emb, WA, bA, WB, bB, WC, bC, WD, bD,
           WE, bE, Wh, bh, Wo, bo):
    src = edge_index[0].astype(jnp.int32)
    dst = edge_index[1].astype(jnp.int32)

    grp = dst // NH
    pos = jnp.zeros((E,), jnp.int32)
    off = jnp.int32(0)
    bounds = []
    for q in range(NPH):
        m = (grp == q)
        cq = jnp.cumsum(m.astype(jnp.int32))
        pos = jnp.where(m, off + cq - 1, pos)
        off = off + cq[E - 1]
        bounds.append(off)
    perm = jnp.zeros((E,), jnp.int32).at[pos].set(
        jnp.arange(E, dtype=jnp.int32))
    src = jnp.take(src, perm)
    dst = jnp.take(dst, perm)
    attr = jnp.take(edge_attr, perm)

    srcA2 = jnp.concatenate([src, src + N])
    dst2 = jnp.concatenate([dst, dst])
    s0v = jnp.concatenate([jnp.full((16,), b, jnp.int32)
                           for b in bounds[:3]])

    h, e2p = _tc_init(x.reshape(N, 1).astype(jnp.int32),
                      attr.reshape(EP, 2), emb)

    for i in range(NL):
        tabA, tabE, hA = _tc_tab(h, WB[i], WD[i], WE[i], bB[i], bD[i],
                                 bE[i], WA[i], bA[i])
        WCAd = jnp.stack([_blkdiag(WC[i][0:H, 0:H]),
                          _blkdiag(WC[i][0:H, H:D])])
        WCBd = jnp.stack([_blkdiag(WC[i][H:D, 0:H]),
                          _blkdiag(WC[i][H:D, H:D])])
        bCp = jnp.concatenate([bC[i].reshape(NC, H)] * 2,
                              axis=1).reshape(NC, 1, D)
        ec2p = _tc_ec(e2p, WCAd, WCBd, bCp)

        enew, numden = _sc_edge_kernel()(e2p.reshape(E, D),
                                         ec2p.reshape(E, D),
                                         tabA, tabE, srcA2, dst2, s0v)
        e2p = enew.reshape(NC, EP, D)
        nd = numden.reshape(NC, NDR, D)
        if i < NL - 1:
            h = _tc_upd(h, hA, nd)
        else:
            out = _tc_final(h, hA, nd, Wh, bh, Wo, bo)
    return out
